# Initial kernel scaffold; baseline (speedup 1.0000x reference)
import jax
import jax.numpy as jnp
from jax import lax
from jax.experimental import pallas as pl
from jax.experimental.pallas import tpu as pltpu
from jax.experimental.pallas import tpu_sc as plsc

M = 4096
NC, NS, L = 2, 16, 16
NW = NC * NS
ROWS_PER_W = M // NW
BATCH = 8


def _body(x_hbm, out_hbm, x_v, y_v, sem):
    c = lax.axis_index("c")
    s = lax.axis_index("s")
    wid = s * NC + c

    pltpu.sync_copy(x_hbm, x_v)

    def build(ci, _):
        v = x_v[pl.ds(ci * L, L)]
        t = ci * L + lax.iota(jnp.int32, L)
        plsc.store_scatter(y_v, [(M - 1) + t], v)
        plsc.store_scatter(y_v, [(M - 1) - t], v)
        return 0

    lax.fori_loop(0, M // L, build, 0)

    row0 = wid * ROWS_PER_W

    def rows(t, _):
        i0 = row0 + t * BATCH
        cps = [
            pltpu.async_copy(
                y_v.at[pl.ds((M - 1) - (i0 + b), M)],
                out_hbm.at[i0 + b],
                sem,
            )
            for b in range(BATCH)
        ]
        for cp in cps:
            cp.wait()
        return 0

    lax.fori_loop(0, ROWS_PER_W // BATCH, rows, 0)


_mesh = plsc.VectorSubcoreMesh(core_axis_name="c", subcore_axis_name="s")

_toeplitz = pl.kernel(
    _body,
    out_type=jax.ShapeDtypeStruct((M, M), jnp.float32),
    mesh=_mesh,
    scratch_types=[
        pltpu.VMEM((M,), jnp.float32),
        pltpu.VMEM((2 * M,), jnp.float32),
        pltpu.SemaphoreType.DMA,
    ],
)


@jax.jit
def kernel(input):
    out = _toeplitz(input.reshape(M).astype(jnp.float32))
    return out[None, :, :]


# speedup vs baseline: 1120.6773x; 1120.6773x over previous
import jax
import jax.numpy as jnp
from jax import lax
from jax.experimental import pallas as pl
from jax.experimental.pallas import tpu as pltpu
from jax.experimental.pallas import tpu_sc as plsc

M = 4096
NC, NS, L = 2, 16, 16
NW = NC * NS
ROWS_PER_W = M // NW
BATCH = 8


def _body(x_hbm, out_hbm, x_v, y_v, sem):
    c = lax.axis_index("c")
    s = lax.axis_index("s")
    wid = s * NC + c

    r = ((M - 1) - wid) % 8
    shift = 8 - r

    pltpu.sync_copy(x_hbm, x_v)

    def build(ci, _):
        v = x_v[pl.ds(ci * L, L)]
        t = ci * L + lax.iota(jnp.int32, L)
        plsc.store_scatter(y_v, [shift + (M - 1) + t], v)
        plsc.store_scatter(y_v, [shift + (M - 1) - t], v)
        return 0

    lax.fori_loop(0, M // L, build, 0)

    def rows(t, _):
        i0 = wid + t * (BATCH * NW)
        cps = [
            pltpu.async_copy(
                y_v.at[pl.ds(pl.multiple_of(shift + (M - 1) - (i0 + b * NW), 8), M)],
                out_hbm.at[i0 + b * NW],
                sem,
            )
            for b in range(BATCH)
        ]
        for cp in cps:
            cp.wait()
        return 0

    lax.fori_loop(0, ROWS_PER_W // BATCH, rows, 0)


_mesh = plsc.VectorSubcoreMesh(core_axis_name="c", subcore_axis_name="s")

_toeplitz = pl.kernel(
    _body,
    out_type=jax.ShapeDtypeStruct((M, M), jnp.float32),
    mesh=_mesh,
    compiler_params=pltpu.CompilerParams(
        needs_layout_passes=False, use_tc_tiling_on_sc=False
    ),
    scratch_types=[
        pltpu.VMEM((M,), jnp.float32),
        pltpu.VMEM((2 * M + 16,), jnp.float32),
        pltpu.SemaphoreType.DMA,
    ],
)


@jax.jit
def kernel(input):
    out = _toeplitz(input.reshape(M).astype(jnp.float32))
    return out[None, :, :]
